# Initial kernel scaffold; baseline (speedup 1.0000x reference)
import jax
import jax.numpy as jnp
from jax.experimental import pallas as pl


def kernel(obj_vecs, pred_vecs, edge_index, params):
    raise NotImplementedError("write your pallas kernel here")


# speedup vs baseline: 2.6918x; 2.6918x over previous
import functools

import jax
import jax.numpy as jnp
from jax import lax
from jax.experimental import pallas as pl
from jax.experimental.pallas import tpu as pltpu
from jax.experimental.pallas import tpu_sc as plsc

N_OBJ = 10000
N_PRED = 160000
H = 512
DOUT = 128

NC = 2
NS = 16
NW = NC * NS

G_EPT = N_PRED // NW
G_CH = 40
G_NCH = G_EPT // G_CH

S_EPT = N_PRED // NS
S_CH = 80
S_NCH = S_EPT // S_CH
SLAB = 128
RPT = 624
RPT_LAST = N_OBJ - (NS - 1) * RPT


def _sc_mesh():
    return plsc.VectorSubcoreMesh(core_axis_name="c", subcore_axis_name="s")


def _gather_body(d, table, sidx, oidx, gs_out, go_out, idx_v, rows_v, sem):
    c = lax.axis_index("c")
    s = lax.axis_index("s")
    wid = s * NC + c
    base = wid * G_EPT

    def run(idx_hbm, out_hbm):
        pltpu.sync_copy(idx_hbm.at[wid], idx_v)

        def step(j, _):
            pltpu.async_copy(table.at[idx_v.at[j]], rows_v, sem).wait()
            pltpu.sync_copy(rows_v, out_hbm.at[pl.ds(base + j * G_CH, G_CH)])
            return ()

        lax.fori_loop(0, G_NCH, step, (), unroll=False)

    run(sidx, gs_out)
    run(oidx, go_out)


def _gather(table, sidx_g, oidx_g):
    d = table.shape[1]
    kfn = pl.kernel(
        functools.partial(_gather_body, d),
        out_type=(
            jax.ShapeDtypeStruct((N_PRED, d), jnp.float32),
            jax.ShapeDtypeStruct((N_PRED, d), jnp.float32),
        ),
        mesh=_sc_mesh(),
        scratch_types=[
            pltpu.VMEM((G_NCH, G_CH), jnp.int32),
            pltpu.VMEM((G_CH, d), jnp.float32),
            pltpu.SemaphoreType.DMA,
        ],
    )
    return kfn(table, sidx_g, oidx_g)


def _fill(ref, rows, cols, value):
    v = jnp.full((16,), value, jnp.float32)

    def zrow(r, _):
        def zcol(k, _):
            ref[r, pl.ds(k * 16, 16)] = v
            return ()
        lax.fori_loop(0, cols // 16, zcol, (), unroll=True)
        return ()

    lax.fori_loop(0, rows, zrow, (), unroll=False)


def _own_rows(s, fn):
    @pl.when(s < NS - 1)
    def _():
        fn(s * RPT, RPT)

    @pl.when(s == NS - 1)
    def _():
        fn((NS - 1) * RPT, RPT_LAST)


def _zero_rows(acc, zbuf, s):
    def do(r0, n):
        def st(i, _):
            pltpu.sync_copy(zbuf, acc.at[pl.ds(r0 + i * 16, 16)])
            return ()
        lax.fori_loop(0, n // 16, st, (), unroll=False)
    _own_rows(s, do)


def _scatter_body(vs, vo, sidx, oidx, out, acc, idx_s, idx_o, vals_v, zbuf):
    c = lax.axis_index("c")
    s = lax.axis_index("s")
    pltpu.sync_copy(sidx.at[s], idx_s)
    pltpu.sync_copy(oidx.at[s], idx_o)
    _fill(zbuf, 16, SLAB, 0.0)

    for p in range(2):
        col0 = c * (2 * SLAB) + p * SLAB
        _zero_rows(acc, zbuf, s)
        plsc.subcore_barrier()
        for idx_v, val_hbm in ((idx_s, vs), (idx_o, vo)):
            def step(j, _):
                pltpu.sync_copy(
                    val_hbm.at[pl.ds(s * S_EPT + j * S_CH, S_CH),
                               pl.ds(col0, SLAB)],
                    vals_v)
                pltpu.sync_copy(vals_v, acc.at[idx_v.at[j]], add=True)
                return ()
            lax.fori_loop(0, S_NCH, step, (), unroll=False)
        plsc.subcore_barrier()
        _own_rows(s, lambda r0, n: pltpu.sync_copy(
            acc.at[pl.ds(r0, n)],
            out.at[pl.ds(r0, n), pl.ds(col0, SLAB)]))


def _scatter(vs, vo, sidx_s, oidx_s):
    kfn = pl.kernel(
        _scatter_body,
        out_type=jax.ShapeDtypeStruct((N_OBJ, H), jnp.float32),
        mesh=_sc_mesh(),
        scratch_types=[
            pltpu.VMEM_SHARED((N_OBJ, SLAB), jnp.float32),
            pltpu.VMEM((S_NCH, S_CH), jnp.int32),
            pltpu.VMEM((S_NCH, S_CH), jnp.int32),
            pltpu.VMEM((S_CH, SLAB), jnp.float32),
            pltpu.VMEM((16, SLAB), jnp.float32),
        ],
    )
    return kfn(vs, vo, sidx_s, oidx_s)


CW = 128


def _counts_body(sidx, oidx, out, acc, idx_v, ones_v, zeros_v):
    c = lax.axis_index("c")
    s = lax.axis_index("s")

    @pl.when(c == 0)
    def _():
        _fill(ones_v, S_CH, CW, 1.0)
        _fill(zeros_v, 16, CW, 0.0)
        _zero_rows(acc, zeros_v, s)
        plsc.subcore_barrier()
        for idx_hbm in (sidx, oidx):
            pltpu.sync_copy(idx_hbm.at[s], idx_v)

            def step(j, _):
                pltpu.sync_copy(ones_v, acc.at[idx_v.at[j]], add=True)
                return ()

            lax.fori_loop(0, S_NCH, step, (), unroll=False)
        plsc.subcore_barrier()
        _own_rows(s, lambda r0, n: pltpu.sync_copy(
            acc.at[pl.ds(r0, n)], out.at[pl.ds(r0, n)]))


def _counts(sidx_s, oidx_s):
    kfn = pl.kernel(
        _counts_body,
        out_type=jax.ShapeDtypeStruct((N_OBJ, CW), jnp.float32),
        mesh=_sc_mesh(),
        scratch_types=[
            pltpu.VMEM_SHARED((N_OBJ, CW), jnp.float32),
            pltpu.VMEM((S_NCH, S_CH), jnp.int32),
            pltpu.VMEM((S_CH, CW), jnp.float32),
            pltpu.VMEM((16, CW), jnp.float32),
        ],
    )
    return kfn(sidx_s, oidx_s)


BE = 800
BN = 1000


def _edge_mlp_body(din, gs, pred, go, w1, b1, w2, b2, ns, np_, no):
    h = jnp.dot(gs[:, :din], w1[:din, :], preferred_element_type=jnp.float32)
    h += jnp.dot(pred[...], w1[din:2 * din, :],
                 preferred_element_type=jnp.float32)
    h += jnp.dot(go[:, :din], w1[2 * din:, :],
                 preferred_element_type=jnp.float32)
    h = jax.nn.relu(h + b1[...])
    ns[...] = jax.nn.relu(
        jnp.dot(h, w2[:, :H], preferred_element_type=jnp.float32)
        + b2[:, :H])
    np_[...] = jax.nn.relu(
        jnp.dot(h, w2[:, H:H + DOUT], preferred_element_type=jnp.float32)
        + b2[:, H:H + DOUT])
    no[...] = jax.nn.relu(
        jnp.dot(h, w2[:, H + DOUT:], preferred_element_type=jnp.float32)
        + b2[:, H + DOUT:])


def _edge_mlp(gs, pred, go, w1, b1, w2, b2):
    din = w1.shape[0] // 3
    dg = gs.shape[1]
    grid = (N_PRED // BE,)
    row = lambda i: (i, 0)
    full = lambda i: (0, 0)
    return pl.pallas_call(
        functools.partial(_edge_mlp_body, din),
        grid=grid,
        in_specs=[
            pl.BlockSpec((BE, dg), row),
            pl.BlockSpec((BE, din), row),
            pl.BlockSpec((BE, dg), row),
            pl.BlockSpec(w1.shape, full),
            pl.BlockSpec(b1.shape, full),
            pl.BlockSpec(w2.shape, full),
            pl.BlockSpec(b2.shape, full),
        ],
        out_specs=[
            pl.BlockSpec((BE, H), row),
            pl.BlockSpec((BE, DOUT), row),
            pl.BlockSpec((BE, H), row),
        ],
        out_shape=[
            jax.ShapeDtypeStruct((N_PRED, H), jnp.float32),
            jax.ShapeDtypeStruct((N_PRED, DOUT), jnp.float32),
            jax.ShapeDtypeStruct((N_PRED, H), jnp.float32),
        ],
    )(gs, pred, go, w1, b1, w2, b2)


def _node_mlp_body(pooled, cnt, w3, b3, w4, b4, out):
    c = cnt[:, 0:1]
    inv = 1.0 / jnp.maximum(c, 1.0)
    h2 = jax.nn.relu(
        jnp.dot(pooled[...] * inv, w3[...], preferred_element_type=jnp.float32)
        + b3[...])
    out[...] = jnp.dot(h2, w4[...], preferred_element_type=jnp.float32) + b4[...]


def _node_mlp(pooled, cnt, w3, b3, w4, b4):
    grid = (N_OBJ // BN,)
    row = lambda i: (i, 0)
    full = lambda i: (0, 0)
    return pl.pallas_call(
        _node_mlp_body,
        grid=grid,
        in_specs=[
            pl.BlockSpec((BN, H), row),
            pl.BlockSpec((BN, CW), row),
            pl.BlockSpec(w3.shape, full),
            pl.BlockSpec(b3.shape, full),
            pl.BlockSpec(w4.shape, full),
            pl.BlockSpec(b4.shape, full),
        ],
        out_specs=pl.BlockSpec((BN, DOUT), row),
        out_shape=jax.ShapeDtypeStruct((N_OBJ, DOUT), jnp.float32),
    )(pooled, cnt, w3, b3, w4, b4)


def kernel(obj_vecs, pred_vecs, edge_index, params):
    s_idx = edge_index[0]
    o_idx = edge_index[1]
    sidx_g = s_idx.reshape(NW, G_NCH, G_CH)
    oidx_g = o_idx.reshape(NW, G_NCH, G_CH)
    sidx_s = s_idx.reshape(NS, S_NCH, S_CH)
    oidx_s = o_idx.reshape(NS, S_NCH, S_CH)

    cnt = _counts(sidx_s, oidx_s)

    ov, pv = obj_vecs, pred_vecs
    for p in params:
        w1, b1, w2, b2, w3, b3, w4, b4 = p
        b1 = b1.reshape(1, -1)
        b2 = b2.reshape(1, -1)
        b3 = b3.reshape(1, -1)
        b4 = b4.reshape(1, -1)
        ovg = ov
        if ovg.shape[1] < DOUT:
            ovg = jnp.pad(ovg, ((0, 0), (0, DOUT - ovg.shape[1])))
        gs, go = _gather(ovg, sidx_g, oidx_g)
        ns, np_, no = _edge_mlp(gs, pv, go, w1, b1, w2, b2)
        pooled = _scatter(ns, no, sidx_s, oidx_s)
        ov = _node_mlp(pooled, cnt, w3, b3, w4, b4)
        pv = np_
    return ov, pv
